# baseline (speedup 1.0000x reference)
import functools

import jax
import jax.numpy as jnp
from jax.experimental import pallas as pl
from jax.experimental.pallas import tpu as pltpu

B, L, D, D_OUT = 16, 2048, 1024, 1024
L2 = L // 2


def _chunk(x_ref, w1_ref, v_ref, mask, lo):
    xb = x_ref[0, 0].astype(jnp.bfloat16)
    h = jnp.tanh(
        jax.lax.dot_general(xb, w1_ref[...], (((1,), (0,)), ((), ())),
                            preferred_element_type=jnp.float32))
    s = jnp.sum(h * v_ref[...], axis=1, keepdims=True)
    p = jnp.exp(s) * mask[lo:lo + L2]
    return xb, p


def _body(x00_ref, x01_ref, x10_ref, x11_ref, mask_ref, w1_ref, v_ref,
          wr_ref, br_ref, out_ref, pool_ref, denom_ref):
    g = pl.program_id(0)

    for r, (xa, xb_) in enumerate(((x00_ref, x01_ref), (x10_ref, x11_ref))):
        mask = mask_ref[r]
        c0, p0 = _chunk(xa, w1_ref, v_ref, mask, 0)
        c1, p1 = _chunk(xb_, w1_ref, v_ref, mask, L2)
        row = g * 2 + r
        denom_ref[pl.ds(row, 1), :] = (jnp.sum(p0) + jnp.sum(p1)).reshape(
            1, 1)
        pooled_u = (
            jax.lax.dot_general(p0.astype(jnp.bfloat16), c0,
                                (((0,), (0,)), ((), ())),
                                preferred_element_type=jnp.float32)
            + jax.lax.dot_general(p1.astype(jnp.bfloat16), c1,
                                  (((0,), (0,)), ((), ())),
                                  preferred_element_type=jnp.float32))
        pool_ref[pl.ds(row, 1), :] = pooled_u

    @pl.when(g == B // 2 - 1)
    def _finish():
        pooled = pool_ref[...] / denom_ref[...]
        out_ref[...] = jax.lax.dot_general(
            pooled, wr_ref[...], (((1,), (0,)), ((), ())),
            preferred_element_type=jnp.float32) + br_ref[...]


@functools.partial(jax.jit, static_argnames=())
def kernel(word_hidden, word_mask, W1, v, Wr, br):
    maskf = word_mask.astype(jnp.float32).reshape(B, L, 1)
    xs = word_hidden.reshape(B, 2, L2, D)
    w1_bf = W1.astype(jnp.bfloat16)
    v2 = v.reshape(1, D)
    br2 = br.reshape(1, D_OUT)

    def xspec(r, i):
        return pl.BlockSpec((1, 1, L2, D),
                            lambda g, r=r, i=i: (2 * g + r, i, 0, 0))

    out = pl.pallas_call(
        _body,
        grid=(B // 2,),
        in_specs=[
            xspec(0, 0), xspec(0, 1), xspec(1, 0), xspec(1, 1),
            pl.BlockSpec((2, L, 1), lambda g: (g, 0, 0)),
            pl.BlockSpec((D, D), lambda g: (0, 0)),
            pl.BlockSpec((1, D), lambda g: (0, 0)),
            pl.BlockSpec((D, D_OUT), lambda g: (0, 0)),
            pl.BlockSpec((1, D_OUT), lambda g: (0, 0)),
        ],
        out_specs=pl.BlockSpec((B, D_OUT), lambda g: (0, 0)),
        out_shape=jax.ShapeDtypeStruct((B, D_OUT), jnp.float32),
        scratch_shapes=[
            pltpu.VMEM((B, D), jnp.float32),
            pltpu.VMEM((B, 1), jnp.float32),
        ],
        compiler_params=pltpu.CompilerParams(
            dimension_semantics=("arbitrary",)),
    )(xs, xs, xs, xs, maskf, w1_bf, v2, Wr, br2)
    return out
